# Initial kernel scaffold; baseline (speedup 1.0000x reference)
import jax
import jax.numpy as jnp
from jax.experimental import pallas as pl


def kernel(psi, task_ids, weights):
    raise NotImplementedError("write your pallas kernel here")


# speedup vs baseline: 1.3966x; 1.3966x over previous
import functools

import jax
import jax.numpy as jnp
from jax import lax
from jax.experimental import pallas as pl
from jax.experimental.pallas import tpu as pltpu
from jax.experimental.pallas import tpu_sc as plsc

B = 16384
D = 128
NW = 32
BPW = B // NW
CHUNK = 128
NCHUNK = BPW // CHUNK
L = 16
UNROLL = 8

_mesh = plsc.VectorSubcoreMesh(core_axis_name="c", subcore_axis_name="s")


@functools.partial(
    pl.kernel,
    mesh=_mesh,
    out_type=jax.ShapeDtypeStruct((B,), jnp.float32),
    scratch_types=[
        pltpu.VMEM((NCHUNK, CHUNK), jnp.int32),
        pltpu.VMEM((CHUNK, D), jnp.float32),
        pltpu.VMEM((CHUNK, D), jnp.float32),
        pltpu.VMEM((CHUNK,), jnp.float32),
        pltpu.VMEM((L * L,), jnp.float32),
        pltpu.SemaphoreType.DMA,
    ],
    compiler_params=pltpu.CompilerParams(needs_layout_passes=False),
)
def _sc_reward_dot(psi_hbm, ids_hbm, w_hbm, out_hbm, idx_v, w_v, psi_v, out_v, t_v, sem):
    wid = lax.axis_index("s") * 2 + lax.axis_index("c")
    base = wid * BPW

    for c in range(NCHUNK):
        pltpu.sync_copy(ids_hbm.at[pl.ds(base + c * CHUNK, CHUNK)], idx_v.at[c])

    for c in range(NCHUNK):
        row0 = base + c * CHUNK
        pltpu.async_copy(w_hbm.at[idx_v.at[c]], w_v, sem).wait()
        pltpu.sync_copy(psi_hbm.at[pl.ds(row0, CHUNK)], psi_v)

        for g in range(CHUNK // L):
            def row_body(r, carry, g=g):
                row = g * L + r
                acc = psi_v[row, pl.ds(0, L)] * w_v[row, pl.ds(0, L)]
                for k in range(1, D // L):
                    acc = acc + psi_v[row, pl.ds(k * L, L)] * w_v[row, pl.ds(k * L, L)]
                plsc.store_scatter(t_v, [lax.iota(jnp.int32, L) * L + r], acc)
                return carry

            lax.fori_loop(0, L, row_body, 0)
            v = t_v[pl.ds(0, L)]
            for l in range(1, L):
                v = v + t_v[pl.ds(l * L, L)]
            out_v[pl.ds(g * L, L)] = v
        pltpu.sync_copy(out_v, out_hbm.at[pl.ds(row0, CHUNK)])


def kernel(psi, task_ids, weights):
    return _sc_reward_dot(psi, task_ids.astype(jnp.int32), weights)


# speedup vs baseline: 1.4454x; 1.0350x over previous
import functools

import jax
import jax.numpy as jnp
from jax import lax
from jax.experimental import pallas as pl
from jax.experimental.pallas import tpu as pltpu
from jax.experimental.pallas import tpu_sc as plsc

B = 16384
D = 128
NW = 32
BPW = B // NW
CHUNK = 128
NCHUNK = BPW // CHUNK
L = 16

_mesh = plsc.VectorSubcoreMesh(core_axis_name="c", subcore_axis_name="s")


@functools.partial(
    pl.kernel,
    mesh=_mesh,
    out_type=jax.ShapeDtypeStruct((B,), jnp.float32),
    scratch_types=[
        pltpu.VMEM((NCHUNK, CHUNK), jnp.int32),
        pltpu.VMEM((2, CHUNK, D), jnp.float32),
        pltpu.VMEM((BPW, D), jnp.float32),
        pltpu.VMEM((BPW,), jnp.float32),
        pltpu.VMEM((L * L,), jnp.float32),
        pltpu.SemaphoreType.DMA,
        pltpu.SemaphoreType.DMA,
        pltpu.SemaphoreType.DMA,
    ],
    compiler_params=pltpu.CompilerParams(needs_layout_passes=False),
)
def _sc_reward_dot(psi_hbm, ids_hbm, w_hbm, out_hbm,
                   idx_v, w_v, psi_v, out_v, t_v, sem_p, sem_w0, sem_w1):
    wid = lax.axis_index("s") * 2 + lax.axis_index("c")
    base = wid * BPW

    for c in range(NCHUNK):
        pltpu.sync_copy(ids_hbm.at[pl.ds(base + c * CHUNK, CHUNK)], idx_v.at[c])

    psi_cp = pltpu.make_async_copy(psi_hbm.at[pl.ds(base, BPW)], psi_v, sem_p)
    psi_cp.start()
    sems = (sem_w0, sem_w1)
    w_cp = [None, None]
    for c in range(2):
        w_cp[c] = pltpu.make_async_copy(w_hbm.at[idx_v.at[c]], w_v.at[c], sems[c])
        w_cp[c].start()
    psi_cp.wait()

    for c in range(NCHUNK):
        b = c & 1
        w_cp[b].wait()
        for g in range(CHUNK // L):
            def row_body(r, carry, b=b, g=g, c=c):
                rowp = c * CHUNK + g * L + r
                roww = g * L + r
                acc = psi_v[rowp, pl.ds(0, L)] * w_v[b, roww, pl.ds(0, L)]
                for k in range(1, D // L):
                    acc = acc + psi_v[rowp, pl.ds(k * L, L)] * w_v[b, roww, pl.ds(k * L, L)]
                plsc.store_scatter(t_v, [lax.iota(jnp.int32, L) * L + r], acc)
                return carry

            lax.fori_loop(0, L, row_body, 0, unroll=2)
            v = t_v[pl.ds(0, L)]
            for l in range(1, L):
                v = v + t_v[pl.ds(l * L, L)]
            out_v[pl.ds(c * CHUNK + g * L, L)] = v
        if c + 2 < NCHUNK:
            w_cp[b] = pltpu.make_async_copy(
                w_hbm.at[idx_v.at[c + 2]], w_v.at[b], sems[b])
            w_cp[b].start()

    pltpu.sync_copy(out_v, out_hbm.at[pl.ds(base, BPW)])


def kernel(psi, task_ids, weights):
    return _sc_reward_dot(psi, task_ids.astype(jnp.int32), weights)


# speedup vs baseline: 1.9151x; 1.3249x over previous
import functools

import jax
import jax.numpy as jnp
from jax import lax
from jax.experimental import pallas as pl
from jax.experimental.pallas import tpu as pltpu
from jax.experimental.pallas import tpu_sc as plsc

B = 16384
D = 128
NW = 32
BPW = B // NW
CHUNK = 128
NCHUNK = BPW // CHUNK
L = 16

_mesh = plsc.VectorSubcoreMesh(core_axis_name="c", subcore_axis_name="s")


@functools.partial(
    pl.kernel,
    mesh=_mesh,
    out_type=jax.ShapeDtypeStruct((B,), jnp.float32),
    scratch_types=[
        pltpu.VMEM((NCHUNK, CHUNK), jnp.int32),
        pltpu.VMEM((2, CHUNK, D), jnp.float32),
        pltpu.VMEM((2, CHUNK, D), jnp.float32),
        pltpu.VMEM((BPW,), jnp.float32),
        pltpu.VMEM((L * L,), jnp.float32),
        pltpu.SemaphoreType.DMA,
        pltpu.SemaphoreType.DMA,
        pltpu.SemaphoreType.DMA,
        pltpu.SemaphoreType.DMA,
        pltpu.SemaphoreType.DMA,
    ],
    compiler_params=pltpu.CompilerParams(needs_layout_passes=False),
)
def _sc_reward_dot(psi_hbm, ids_hbm, w_hbm, out_hbm,
                   idx_v, w_v, psi_v, out_v, t_v,
                   sem_i, sem_p0, sem_p1, sem_w0, sem_w1):
    wid = lax.axis_index("s") * 2 + lax.axis_index("c")
    base = wid * BPW
    sems_p = (sem_p0, sem_p1)
    sems_w = (sem_w0, sem_w1)

    id_cps = []
    for c in range(NCHUNK):
        cp = pltpu.make_async_copy(
            ids_hbm.at[pl.ds(base + c * CHUNK, CHUNK)], idx_v.at[c], sem_i)
        cp.start()
        id_cps.append(cp)
    for cp in id_cps:
        cp.wait()

    def start_chunk(c):
        b = c & 1
        p = pltpu.make_async_copy(
            psi_hbm.at[pl.ds(base + c * CHUNK, CHUNK)], psi_v.at[b], sems_p[b])
        p.start()
        w = pltpu.make_async_copy(w_hbm.at[idx_v.at[c]], w_v.at[b], sems_w[b])
        w.start()
        return p, w

    cps = [start_chunk(0), start_chunk(1)]

    for c in range(NCHUNK):
        b = c & 1
        p_cp, w_cp = cps[b]
        p_cp.wait()
        w_cp.wait()

        def group_body(g, carry, b=b, c=c):
            def row_body(r, carry2, b=b, g=g):
                row = g * L + r
                acc = psi_v[b, row, pl.ds(0, L)] * w_v[b, row, pl.ds(0, L)]
                for k in range(1, D // L):
                    acc = acc + psi_v[b, row, pl.ds(k * L, L)] * w_v[b, row, pl.ds(k * L, L)]
                plsc.store_scatter(t_v, [lax.iota(jnp.int32, L) * L + r], acc)
                return carry2

            lax.fori_loop(0, L, row_body, 0, unroll=2)
            v = t_v[pl.ds(0, L)]
            for l in range(1, L):
                v = v + t_v[pl.ds(l * L, L)]
            out_v[pl.ds(c * CHUNK + g * L, L)] = v
            return carry

        lax.fori_loop(0, CHUNK // L, group_body, 0)
        if c + 2 < NCHUNK:
            cps[b] = start_chunk(c + 2)

    pltpu.sync_copy(out_v, out_hbm.at[pl.ds(base, BPW)])


def kernel(psi, task_ids, weights):
    return _sc_reward_dot(psi, task_ids.astype(jnp.int32), weights)


# speedup vs baseline: 1.9913x; 1.0398x over previous
import functools

import jax
import jax.numpy as jnp
from jax import lax
from jax.experimental import pallas as pl
from jax.experimental.pallas import tpu as pltpu
from jax.experimental.pallas import tpu_sc as plsc

B = 16384
D = 128
V = 1000
NW = 32
NS = 16
BPW = B // NW
CHUNK = 128
NCHUNK = BPW // CHUNK
L = 16
NLOADERS = 8
VROWS = V // NLOADERS

_mesh = plsc.VectorSubcoreMesh(core_axis_name="c", subcore_axis_name="s")


@functools.partial(
    pl.kernel,
    mesh=_mesh,
    out_type=jax.ShapeDtypeStruct((B,), jnp.float32),
    scratch_types=[
        pltpu.VMEM((NCHUNK, CHUNK), jnp.int32),
        pltpu.VMEM((2, CHUNK, D), jnp.float32),
        pltpu.VMEM((2, CHUNK, D), jnp.float32),
        pltpu.VMEM((BPW,), jnp.float32),
        pltpu.VMEM((L * L,), jnp.float32),
        pltpu.VMEM_SHARED((V, D), jnp.float32),
        pltpu.SemaphoreType.DMA,
        pltpu.SemaphoreType.DMA,
        pltpu.SemaphoreType.DMA,
        pltpu.SemaphoreType.DMA,
        pltpu.SemaphoreType.DMA,
        pltpu.SemaphoreType.DMA,
    ],
    compiler_params=pltpu.CompilerParams(needs_layout_passes=False),
)
def _sc_reward_dot(psi_hbm, ids_hbm, w_hbm, out_hbm,
                   idx_v, w_v, psi_v, out_v, t_v, tbl_s,
                   sem_t, sem_i, sem_p0, sem_p1, sem_w0, sem_w1):
    sid = lax.axis_index("s")
    wid = sid * 2 + lax.axis_index("c")
    base = wid * BPW
    sems_p = (sem_p0, sem_p1)
    sems_w = (sem_w0, sem_w1)

    id_cps = []
    for c in range(NCHUNK):
        cp = pltpu.make_async_copy(
            ids_hbm.at[pl.ds(base + c * CHUNK, CHUNK)], idx_v.at[c], sem_i)
        cp.start()
        id_cps.append(cp)

    for t in range(NLOADERS):
        r0 = t * 128
        nr = min(128, V - r0)

        @pl.when(sid == t)
        def _load_table(r0=r0, nr=nr):
            tcp = pltpu.make_async_copy(
                w_hbm.at[pl.ds(r0, nr)], tbl_s.at[pl.ds(r0, nr)], sem_t)
            tcp.start()
            tcp.wait()

    def start_psi(c):
        b = c & 1
        p = pltpu.make_async_copy(
            psi_hbm.at[pl.ds(base + c * CHUNK, CHUNK)], psi_v.at[b], sems_p[b])
        p.start()
        return p

    psi_cps = [start_psi(0), start_psi(1)]
    for cp in id_cps:
        cp.wait()
    plsc.subcore_barrier()

    def start_w(c):
        b = c & 1
        w = pltpu.make_async_copy(tbl_s.at[idx_v.at[c]], w_v.at[b], sems_w[b])
        w.start()
        return w

    cps = [(psi_cps[0], start_w(0)), (psi_cps[1], start_w(1))]

    for c in range(NCHUNK):
        b = c & 1
        p_cp, w_cp = cps[b]
        p_cp.wait()
        w_cp.wait()

        def group_body(g, carry, b=b, c=c):
            def row_body(r, carry2, b=b, g=g):
                row = g * L + r
                acc = psi_v[b, row, pl.ds(0, L)] * w_v[b, row, pl.ds(0, L)]
                for k in range(1, D // L):
                    acc = acc + psi_v[b, row, pl.ds(k * L, L)] * w_v[b, row, pl.ds(k * L, L)]
                plsc.store_scatter(t_v, [lax.iota(jnp.int32, L) * L + r], acc)
                return carry2

            lax.fori_loop(0, L, row_body, 0, unroll=2)
            v = t_v[pl.ds(0, L)]
            for l in range(1, L):
                v = v + t_v[pl.ds(l * L, L)]
            out_v[pl.ds(c * CHUNK + g * L, L)] = v
            return carry

        lax.fori_loop(0, CHUNK // L, group_body, 0)
        if c + 2 < NCHUNK:
            cps[b] = (start_psi(c + 2), start_w(c + 2))

    pltpu.sync_copy(out_v, out_hbm.at[pl.ds(base, BPW)])


def kernel(psi, task_ids, weights):
    return _sc_reward_dot(psi, task_ids.astype(jnp.int32), weights)
